# Initial kernel scaffold; baseline (speedup 1.0000x reference)
import functools

import jax
import jax.numpy as jnp
from jax import lax
from jax.experimental import pallas as pl
from jax.experimental.pallas import tpu as pltpu
from jax.experimental.pallas import tpu_sc as plsc

N = 10000
E = 320000
F_IN = 128
H = 64
OUT = 128
G = 16

NC, NS, L = 2, 16, 16
NW = NC * NS
PT = 640
NPAD = NS * PT
CH = 128
EC = 79
PE = NW * EC * CH

_f32 = jnp.float32
_mesh = plsc.VectorSubcoreMesh(core_axis_name="c", subcore_axis_name="s")


@functools.partial(
    pl.kernel,
    out_type=jax.ShapeDtypeStruct((NC, NPAD), _f32),
    mesh=_mesh,
    scratch_types=[
        pltpu.VMEM((EC, CH), jnp.int32),
        pltpu.VMEM((CH,), _f32),
        pltpu.VMEM((PT,), _f32),
        pltpu.VMEM_SHARED((NPAD,), _f32),
    ],
)
def _deg_kernel(dst_hbm, out_hbm, idx_d, ones_v, zbuf, acc):
    c = lax.axis_index("c")
    s = lax.axis_index("s")
    wid = c * NS + s
    pltpu.sync_copy(dst_hbm.at[wid], idx_d)
    for i in range(CH // L):
        ones_v[pl.ds(i * L, L)] = jnp.ones((L,), _f32)

    @pl.loop(0, PT // L)
    def _(i):
        zbuf[pl.ds(i * L, L)] = jnp.zeros((L,), _f32)

    pltpu.sync_copy(zbuf, acc.at[pl.ds(s * PT, PT)])
    plsc.subcore_barrier()

    @pl.loop(0, EC)
    def _(j):
        pltpu.sync_copy(ones_v, acc.at[idx_d.at[j]], add=True)

    plsc.subcore_barrier()
    pltpu.sync_copy(acc.at[pl.ds(s * PT, PT)], out_hbm.at[c, pl.ds(s * PT, PT)])


@functools.partial(
    pl.kernel,
    out_type=jax.ShapeDtypeStruct((NC, NPAD, H), _f32),
    mesh=_mesh,
    scratch_types=[
        pltpu.VMEM((EC, CH), jnp.int32),
        pltpu.VMEM((EC, CH), jnp.int32),
        pltpu.VMEM((CH, H), _f32),
        pltpu.VMEM((CH, H), _f32),
        pltpu.VMEM_SHARED((NPAD, H), _f32),
        pltpu.SemaphoreType.DMA,
        pltpu.SemaphoreType.DMA,
    ],
)
def _mp_kernel(y_hbm, src_hbm, dst_hbm, out_hbm,
               idx_s, idx_d, buf_a, buf_b, acc, sem_a, sem_b):
    c = lax.axis_index("c")
    s = lax.axis_index("s")
    wid = c * NS + s
    pltpu.sync_copy(src_hbm.at[wid], idx_s)
    pltpu.sync_copy(dst_hbm.at[wid], idx_d)

    @pl.when(c == 0)
    def _():
        pltpu.sync_copy(y_hbm.at[pl.ds(s * PT, PT)], acc.at[pl.ds(s * PT, PT)])

    @pl.when(c == 1)
    def _():
        @pl.loop(0, CH)
        def _(i):
            for jj in range(H // L):
                buf_a[i, pl.ds(jj * L, L)] = jnp.zeros((L,), _f32)

        @pl.loop(0, PT // CH)
        def _(k):
            pltpu.sync_copy(buf_a, acc.at[pl.ds(s * PT + k * CH, CH)])

    plsc.subcore_barrier()

    @pl.loop(0, EC)
    def _(j):
        pltpu.async_copy(y_hbm.at[idx_s.at[j]], buf_b, sem_b).wait()
        pltpu.sync_copy(buf_b, acc.at[idx_d.at[j]], add=True)

    plsc.subcore_barrier()
    pltpu.sync_copy(acc.at[pl.ds(s * PT, PT)], out_hbm.at[c, pl.ds(s * PT, PT)])


_BT = 2048


def _tc1_body(d0_ref, d1_ref, x_ref, w_ref, dis_ref, y_ref):
    i = pl.program_id(0)
    deg = d0_ref[...] + d1_ref[...] + 1.0
    rows = lax.broadcasted_iota(jnp.int32, (_BT, 1), 0) + i * _BT
    dis = jnp.where(rows < N, lax.rsqrt(deg), 0.0)
    dis_ref[...] = dis
    y_ref[...] = jnp.dot(x_ref[...], w_ref[...],
                         preferred_element_type=_f32) * dis


def _tc1_call(d0, d1, x_p, w1):
    return pl.pallas_call(
        _tc1_body,
        grid=(NPAD // _BT,),
        in_specs=[
            pl.BlockSpec((_BT, 1), lambda i: (i, 0)),
            pl.BlockSpec((_BT, 1), lambda i: (i, 0)),
            pl.BlockSpec((_BT, F_IN), lambda i: (i, 0)),
            pl.BlockSpec((F_IN, H), lambda i: (0, 0)),
        ],
        out_specs=[
            pl.BlockSpec((_BT, 1), lambda i: (i, 0)),
            pl.BlockSpec((_BT, H), lambda i: (i, 0)),
        ],
        out_shape=[
            jax.ShapeDtypeStruct((NPAD, 1), _f32),
            jax.ShapeDtypeStruct((NPAD, H), _f32),
        ],
    )(d0, d1, x_p, w1)


def _tc2_body(a0_ref, a1_ref, dis_ref, b_ref, w_ref, y_ref):
    dis = dis_ref[...]
    h = jax.nn.relu(dis * (a0_ref[...] + a1_ref[...]) + b_ref[...])
    y_ref[...] = jnp.dot(h, w_ref[...], preferred_element_type=_f32) * dis


def _tc2_call(a0, a1, dis, b, w):
    return pl.pallas_call(
        _tc2_body,
        grid=(NPAD // _BT,),
        in_specs=[
            pl.BlockSpec((_BT, H), lambda i: (i, 0)),
            pl.BlockSpec((_BT, H), lambda i: (i, 0)),
            pl.BlockSpec((_BT, 1), lambda i: (i, 0)),
            pl.BlockSpec((1, H), lambda i: (0, 0)),
            pl.BlockSpec((H, H), lambda i: (0, 0)),
        ],
        out_specs=pl.BlockSpec((_BT, H), lambda i: (i, 0)),
        out_shape=jax.ShapeDtypeStruct((NPAD, H), _f32),
    )(a0, a1, dis, b, w)


def _tcf_body(a0_ref, a1_ref, dis_ref, b3_ref, batch_ref, wl_ref, bl_ref,
              out_ref, sums, cnts):
    i = pl.program_id(0)

    @pl.when(i == 0)
    def _():
        sums[...] = jnp.zeros_like(sums)
        cnts[...] = jnp.zeros_like(cnts)

    h = dis_ref[...] * (a0_ref[...] + a1_ref[...]) + b3_ref[...]
    bt = batch_ref[...]
    onehot = (lax.broadcasted_iota(jnp.int32, (G, _BT), 0) == bt).astype(_f32)
    sums[...] += jnp.dot(onehot, h, preferred_element_type=_f32)
    cnts[...] += jnp.sum(onehot, axis=1, keepdims=True)

    @pl.when(i == pl.num_programs(0) - 1)
    def _():
        pooled = sums[...] / jnp.maximum(cnts[...], 1.0)
        out_ref[...] = jnp.dot(pooled, wl_ref[...],
                               preferred_element_type=_f32) + bl_ref[...]


def _tcf_call(a0, a1, dis, b3, batch_p, wl, bl):
    return pl.pallas_call(
        _tcf_body,
        grid=(NPAD // _BT,),
        in_specs=[
            pl.BlockSpec((_BT, H), lambda i: (i, 0)),
            pl.BlockSpec((_BT, H), lambda i: (i, 0)),
            pl.BlockSpec((_BT, 1), lambda i: (i, 0)),
            pl.BlockSpec((1, H), lambda i: (0, 0)),
            pl.BlockSpec((1, _BT), lambda i: (0, i)),
            pl.BlockSpec((H, OUT), lambda i: (0, 0)),
            pl.BlockSpec((1, OUT), lambda i: (0, 0)),
        ],
        out_specs=pl.BlockSpec((G, OUT), lambda i: (0, 0)),
        out_shape=jax.ShapeDtypeStruct((G, OUT), _f32),
        scratch_shapes=[
            pltpu.VMEM((G, H), _f32),
            pltpu.VMEM((G, 1), _f32),
        ],
    )(a0, a1, dis, b3, batch_p, wl, bl)


def kernel(x, edge_index, batch, W1, b1, W2, b2, W3, b3, Wl, bl):
    pad_e = PE - E
    src_p = jnp.concatenate(
        [edge_index[0], jnp.zeros((pad_e,), jnp.int32)]).reshape(NW, EC, CH)
    dst_p = jnp.concatenate(
        [edge_index[1], jnp.full((pad_e,), N, jnp.int32)]).reshape(NW, EC, CH)
    x_p = jnp.pad(x, ((0, NPAD - N), (0, 0)))
    batch_p = jnp.concatenate(
        [batch, jnp.full((NPAD - N,), G, jnp.int32)]).reshape(1, NPAD)

    degs = _deg_kernel(dst_p)
    d0 = degs[0].reshape(NPAD, 1)
    d1 = degs[1].reshape(NPAD, 1)
    dis, y1 = _tc1_call(d0, d1, x_p, W1)

    a = _mp_kernel(y1, src_p, dst_p)
    y2 = _tc2_call(a[0], a[1], dis, b1.reshape(1, H), W2)
    a = _mp_kernel(y2, src_p, dst_p)
    y3 = _tc2_call(a[0], a[1], dis, b2.reshape(1, H), W3)
    a = _mp_kernel(y3, src_p, dst_p)
    return _tcf_call(a[0], a[1], dis, b3.reshape(1, H), batch_p,
                     Wl, bl.reshape(1, OUT))


# speedup vs baseline: 16.4400x; 16.4400x over previous
import functools

import jax
import jax.numpy as jnp
from jax import lax
from jax.experimental import pallas as pl
from jax.experimental.pallas import tpu as pltpu
from jax.experimental.pallas import tpu_sc as plsc

N = 10000
E = 320000
F_IN = 128
H = 64
OUT = 128
G = 16

NC, NS, L = 2, 16, 16
NW = NC * NS
PT = 640
NPAD = NS * PT
CH = 128
EC = 79
PE = NW * EC * CH

_f32 = jnp.float32
_mesh = plsc.VectorSubcoreMesh(core_axis_name="c", subcore_axis_name="s")
_sc_params = pltpu.CompilerParams(use_tc_tiling_on_sc=False)


@functools.partial(
    pl.kernel,
    out_type=jax.ShapeDtypeStruct((NC, NPAD), _f32),
    mesh=_mesh,
    compiler_params=_sc_params,
    scratch_types=[
        pltpu.VMEM((EC, CH), jnp.int32),
        pltpu.VMEM((CH,), _f32),
        pltpu.VMEM((PT,), _f32),
        pltpu.VMEM_SHARED((NPAD,), _f32),
    ],
)
def _deg_kernel(dst_hbm, out_hbm, idx_d, ones_v, zbuf, acc):
    c = lax.axis_index("c")
    s = lax.axis_index("s")
    wid = c * NS + s
    pltpu.sync_copy(dst_hbm.at[wid], idx_d)
    for i in range(CH // L):
        ones_v[pl.ds(i * L, L)] = jnp.ones((L,), _f32)

    @pl.loop(0, PT // L)
    def _(i):
        zbuf[pl.ds(i * L, L)] = jnp.zeros((L,), _f32)

    pltpu.sync_copy(zbuf, acc.at[pl.ds(s * PT, PT)])
    plsc.subcore_barrier()

    @pl.loop(0, EC)
    def _(j):
        pltpu.sync_copy(ones_v, acc.at[idx_d.at[j]], add=True)

    plsc.subcore_barrier()
    pltpu.sync_copy(acc.at[pl.ds(s * PT, PT)], out_hbm.at[c, pl.ds(s * PT, PT)])


@functools.partial(
    pl.kernel,
    out_type=jax.ShapeDtypeStruct((NC, NPAD, H), _f32),
    mesh=_mesh,
    compiler_params=_sc_params,
    scratch_types=[
        pltpu.VMEM((EC, CH), jnp.int32),
        pltpu.VMEM((EC, CH), jnp.int32),
        pltpu.VMEM((CH, H), _f32),
        pltpu.VMEM((CH, H), _f32),
        pltpu.VMEM_SHARED((NPAD, H), _f32),
        pltpu.SemaphoreType.DMA,
        pltpu.SemaphoreType.DMA,
    ],
)
def _mp_kernel(y_hbm, src_hbm, dst_hbm, out_hbm,
               idx_s, idx_d, buf_a, buf_b, acc, sem_a, sem_b):
    c = lax.axis_index("c")
    s = lax.axis_index("s")
    wid = c * NS + s
    pltpu.sync_copy(src_hbm.at[wid], idx_s)
    pltpu.sync_copy(dst_hbm.at[wid], idx_d)

    @pl.when(c == 0)
    def _():
        pltpu.sync_copy(y_hbm.at[pl.ds(s * PT, PT)], acc.at[pl.ds(s * PT, PT)])

    @pl.when(c == 1)
    def _():
        @pl.loop(0, CH)
        def _(i):
            for jj in range(H // L):
                buf_a[i, pl.ds(jj * L, L)] = jnp.zeros((L,), _f32)

        @pl.loop(0, PT // CH)
        def _(k):
            pltpu.sync_copy(buf_a, acc.at[pl.ds(s * PT + k * CH, CH)])

    plsc.subcore_barrier()

    @pl.loop(0, EC)
    def _(j):
        pltpu.async_copy(y_hbm.at[idx_s.at[j]], buf_b, sem_b).wait()
        pltpu.sync_copy(buf_b, acc.at[idx_d.at[j]], add=True)

    plsc.subcore_barrier()
    pltpu.sync_copy(acc.at[pl.ds(s * PT, PT)], out_hbm.at[c, pl.ds(s * PT, PT)])


_BT = 2048


def _tc1_body(d0_ref, d1_ref, x_ref, w_ref, dis_ref, y_ref):
    i = pl.program_id(0)
    deg = d0_ref[...] + d1_ref[...] + 1.0
    rows = lax.broadcasted_iota(jnp.int32, (_BT, 1), 0) + i * _BT
    dis = jnp.where(rows < N, lax.rsqrt(deg), 0.0)
    dis_ref[...] = dis
    y_ref[...] = jnp.dot(x_ref[...], w_ref[...],
                         preferred_element_type=_f32) * dis


def _tc1_call(d0, d1, x_p, w1):
    return pl.pallas_call(
        _tc1_body,
        grid=(NPAD // _BT,),
        in_specs=[
            pl.BlockSpec((_BT, 1), lambda i: (i, 0)),
            pl.BlockSpec((_BT, 1), lambda i: (i, 0)),
            pl.BlockSpec((_BT, F_IN), lambda i: (i, 0)),
            pl.BlockSpec((F_IN, H), lambda i: (0, 0)),
        ],
        out_specs=[
            pl.BlockSpec((_BT, 1), lambda i: (i, 0)),
            pl.BlockSpec((_BT, H), lambda i: (i, 0)),
        ],
        out_shape=[
            jax.ShapeDtypeStruct((NPAD, 1), _f32),
            jax.ShapeDtypeStruct((NPAD, H), _f32),
        ],
    )(d0, d1, x_p, w1)


def _tc2_body(a0_ref, a1_ref, dis_ref, b_ref, w_ref, y_ref):
    dis = dis_ref[...]
    h = jax.nn.relu(dis * (a0_ref[...] + a1_ref[...]) + b_ref[...])
    y_ref[...] = jnp.dot(h, w_ref[...], preferred_element_type=_f32) * dis


def _tc2_call(a0, a1, dis, b, w):
    return pl.pallas_call(
        _tc2_body,
        grid=(NPAD // _BT,),
        in_specs=[
            pl.BlockSpec((_BT, H), lambda i: (i, 0)),
            pl.BlockSpec((_BT, H), lambda i: (i, 0)),
            pl.BlockSpec((_BT, 1), lambda i: (i, 0)),
            pl.BlockSpec((1, H), lambda i: (0, 0)),
            pl.BlockSpec((H, H), lambda i: (0, 0)),
        ],
        out_specs=pl.BlockSpec((_BT, H), lambda i: (i, 0)),
        out_shape=jax.ShapeDtypeStruct((NPAD, H), _f32),
    )(a0, a1, dis, b, w)


def _tcf_body(a0_ref, a1_ref, dis_ref, b3_ref, batch_ref, wl_ref, bl_ref,
              out_ref, sums, cnts):
    i = pl.program_id(0)

    @pl.when(i == 0)
    def _():
        sums[...] = jnp.zeros_like(sums)
        cnts[...] = jnp.zeros_like(cnts)

    h = dis_ref[...] * (a0_ref[...] + a1_ref[...]) + b3_ref[...]
    bt = batch_ref[...]
    onehot = (lax.broadcasted_iota(jnp.int32, (G, _BT), 0) == bt).astype(_f32)
    sums[...] += jnp.dot(onehot, h, preferred_element_type=_f32)
    cnts[...] += jnp.sum(onehot, axis=1, keepdims=True)

    @pl.when(i == pl.num_programs(0) - 1)
    def _():
        pooled = sums[...] / jnp.maximum(cnts[...], 1.0)
        out_ref[...] = jnp.dot(pooled, wl_ref[...],
                               preferred_element_type=_f32) + bl_ref[...]


def _tcf_call(a0, a1, dis, b3, batch_p, wl, bl):
    return pl.pallas_call(
        _tcf_body,
        grid=(NPAD // _BT,),
        in_specs=[
            pl.BlockSpec((_BT, H), lambda i: (i, 0)),
            pl.BlockSpec((_BT, H), lambda i: (i, 0)),
            pl.BlockSpec((_BT, 1), lambda i: (i, 0)),
            pl.BlockSpec((1, H), lambda i: (0, 0)),
            pl.BlockSpec((1, _BT), lambda i: (0, i)),
            pl.BlockSpec((H, OUT), lambda i: (0, 0)),
            pl.BlockSpec((1, OUT), lambda i: (0, 0)),
        ],
        out_specs=pl.BlockSpec((G, OUT), lambda i: (0, 0)),
        out_shape=jax.ShapeDtypeStruct((G, OUT), _f32),
        scratch_shapes=[
            pltpu.VMEM((G, H), _f32),
            pltpu.VMEM((G, 1), _f32),
        ],
    )(a0, a1, dis, b3, batch_p, wl, bl)


def kernel(x, edge_index, batch, W1, b1, W2, b2, W3, b3, Wl, bl):
    pad_e = PE - E
    src_p = jnp.concatenate(
        [edge_index[0], jnp.zeros((pad_e,), jnp.int32)]).reshape(NW, EC, CH)
    dst_p = jnp.concatenate(
        [edge_index[1], jnp.full((pad_e,), N, jnp.int32)]).reshape(NW, EC, CH)
    x_p = jnp.pad(x, ((0, NPAD - N), (0, 0)))
    batch_p = jnp.concatenate(
        [batch, jnp.full((NPAD - N,), G, jnp.int32)]).reshape(1, NPAD)

    degs = _deg_kernel(dst_p)
    d0 = degs[0].reshape(NPAD, 1)
    d1 = degs[1].reshape(NPAD, 1)
    dis, y1 = _tc1_call(d0, d1, x_p, W1)

    a = _mp_kernel(y1, src_p, dst_p)
    y2 = _tc2_call(a[0], a[1], dis, b1.reshape(1, H), W2)
    a = _mp_kernel(y2, src_p, dst_p)
    y3 = _tc2_call(a[0], a[1], dis, b2.reshape(1, H), W3)
    a = _mp_kernel(y3, src_p, dst_p)
    return _tcf_call(a[0], a[1], dis, b3.reshape(1, H), batch_p,
                     Wl, bl.reshape(1, OUT))
